# Initial kernel scaffold; baseline (speedup 1.0000x reference)
import jax
import jax.numpy as jnp
from jax.experimental import pallas as pl


def kernel(x, table):
    raise NotImplementedError("write your pallas kernel here")


# speedup vs baseline: 1.7475x; 1.7475x over previous
import jax
import jax.numpy as jnp
from jax.experimental import pallas as pl
from jax.experimental.pallas import tpu as pltpu
from jax.experimental.pallas import tpu_sc as plsc

_EMBED_DIM = 64
_WINDOW = 128


def _gather_rows(table, idx_flat):
    num_idx = idx_flat.shape[0]
    idx2d = idx_flat.reshape(1, num_idx)
    mesh = plsc.VectorSubcoreMesh(
        core_axis_name="core", subcore_axis_name="subcore"
    )

    @pl.kernel(
        out_type=jax.ShapeDtypeStruct((num_idx, _EMBED_DIM), table.dtype),
        mesh=mesh,
        compiler_params=pltpu.CompilerParams(use_tc_tiling_on_sc=False),
    )
    def k(table_hbm, i_hbm, o_hbm):
        def body(i_vmem, o_vmem):
            pltpu.sync_copy(table_hbm.at[i_vmem.at[0]], o_vmem)

        pltpu.emit_pipeline(
            body,
            grid=(num_idx // _WINDOW,),
            in_specs=[pl.BlockSpec((1, _WINDOW), index_map=lambda i: (0, i))],
            out_specs=[
                pl.BlockSpec((_WINDOW, _EMBED_DIM), index_map=lambda i: (i, 0))
            ],
            core_axis_name=("core", "subcore"),
            dimension_semantics=(pltpu.PARALLEL,),
        )(i_hbm, o_hbm)

    return k(table, idx2d)


def kernel(x, table):
    batch, hist = x.shape
    idx_flat = x.astype(jnp.int32).reshape(-1)
    rows = _gather_rows(table, idx_flat)
    return rows.reshape(batch, hist, _EMBED_DIM)


# speedup vs baseline: 1.8705x; 1.0704x over previous
import jax
import jax.numpy as jnp
from jax.experimental import pallas as pl
from jax.experimental.pallas import tpu as pltpu
from jax.experimental.pallas import tpu_sc as plsc

_EMBED_DIM = 64
_WINDOW = 512


def _gather_rows(table, idx_flat):
    num_idx = idx_flat.shape[0]
    idx2d = idx_flat.reshape(1, num_idx)
    mesh = plsc.VectorSubcoreMesh(
        core_axis_name="core", subcore_axis_name="subcore"
    )

    @pl.kernel(
        out_type=jax.ShapeDtypeStruct((num_idx, _EMBED_DIM), table.dtype),
        mesh=mesh,
        compiler_params=pltpu.CompilerParams(use_tc_tiling_on_sc=False),
    )
    def k(table_hbm, i_hbm, o_hbm):
        def body(i_vmem, o_vmem):
            pltpu.sync_copy(table_hbm.at[i_vmem.at[0]], o_vmem)

        pltpu.emit_pipeline(
            body,
            grid=(num_idx // _WINDOW,),
            in_specs=[pl.BlockSpec((1, _WINDOW), index_map=lambda i: (0, i))],
            out_specs=[
                pl.BlockSpec((_WINDOW, _EMBED_DIM), index_map=lambda i: (i, 0))
            ],
            core_axis_name=("core", "subcore"),
            dimension_semantics=(pltpu.PARALLEL,),
        )(i_hbm, o_hbm)

    return k(table, idx2d)


def kernel(x, table):
    batch, hist = x.shape
    idx_flat = x.astype(jnp.int32).reshape(-1)
    rows = _gather_rows(table, idx_flat)
    return rows.reshape(batch, hist, _EMBED_DIM)


# speedup vs baseline: 1.8767x; 1.0033x over previous
import jax
import jax.numpy as jnp
from jax import lax
from jax.experimental import pallas as pl
from jax.experimental.pallas import tpu as pltpu
from jax.experimental.pallas import tpu_sc as plsc

_EMBED_DIM = 64
_NBUF = 4
_CHUNK = 320


def _gather_rows(table, idx_flat):
    num_idx = idx_flat.shape[0]
    info = plsc.get_sparse_core_info()
    nw = info.num_cores * info.num_subcores
    per_w = num_idx // nw
    nchunks = per_w // _CHUNK
    assert per_w % _CHUNK == 0 and nchunks % _NBUF == 0 and nchunks >= 4
    mesh = plsc.VectorSubcoreMesh(
        core_axis_name="core", subcore_axis_name="subcore"
    )

    @pl.kernel(
        out_type=jax.ShapeDtypeStruct((num_idx, _EMBED_DIM), table.dtype),
        mesh=mesh,
        compiler_params=pltpu.CompilerParams(use_tc_tiling_on_sc=False),
        scratch_types=[
            pltpu.VMEM((per_w,), jnp.int32),
            pltpu.VMEM((_NBUF, _CHUNK, _EMBED_DIM), jnp.float32),
        ]
        + [pltpu.SemaphoreType.DMA] * (2 * _NBUF),
    )
    def k(table_hbm, idx_hbm, out_hbm, idx_v, rows_v, *sems):
        gsem, osem = sems[:_NBUF], sems[_NBUF:]
        wid = lax.axis_index("subcore") * info.num_cores + lax.axis_index(
            "core"
        )
        base = wid * per_w
        pltpu.sync_copy(idx_hbm.at[pl.ds(base, per_w)], idx_v)

        def g_copy(i, slot):
            return pltpu.make_async_copy(
                table_hbm.at[idx_v.at[pl.ds(i * _CHUNK, _CHUNK)]],
                rows_v.at[slot],
                gsem[slot],
            )

        def o_copy(i, slot):
            return pltpu.make_async_copy(
                rows_v.at[slot],
                out_hbm.at[pl.ds(base + i * _CHUNK, _CHUNK)],
                osem[slot],
            )

        g_copy(0, 0).start()
        g_copy(1, 1).start()

        @pl.loop(0, nchunks // _NBUF)
        def _(g):
            for b in range(_NBUF):
                i = g * _NBUF + b
                s2 = (b + 2) % _NBUF
                g_copy(i, b).wait()
                o_copy(i, b).start()

                @pl.when(i >= 2)
                def _():
                    o_copy(i - 2, s2).wait()

                @pl.when(i + 2 < nchunks)
                def _():
                    g_copy(i + 2, s2).start()

        o_copy(nchunks - 2, (nchunks - 2) % _NBUF).wait()
        o_copy(nchunks - 1, (nchunks - 1) % _NBUF).wait()

    return k(table, idx_flat)


def kernel(x, table):
    batch, hist = x.shape
    idx_flat = x.astype(jnp.int32).reshape(-1)
    rows = _gather_rows(table, idx_flat)
    return rows.reshape(batch, hist, _EMBED_DIM)
